# Initial kernel scaffold; baseline (speedup 1.0000x reference)
import jax
import jax.numpy as jnp
from jax.experimental import pallas as pl


def kernel(feature, edge_index):
    raise NotImplementedError("write your pallas kernel here")


# speedup vs baseline: 4.7478x; 4.7478x over previous
import functools

import jax
import jax.numpy as jnp
from jax import lax
from jax.experimental import pallas as pl
from jax.experimental.pallas import tpu as pltpu
from jax.experimental.pallas import tpu_sc as plsc

N = 10000
D = 128
K = 8
NP = 10240
NC, NS, L = 2, 16, 16
NW = NC * NS
CHUNK = 128
CPT = 80
EP = NW * CPT * CHUNK
RPS = NP // NS
BLK = 256


def _mesh():
    return plsc.VectorSubcoreMesh(core_axis_name="c", subcore_axis_name="s")


def _sc_hist(rows2d, colp2d):
    @functools.partial(
        pl.kernel,
        out_type=jax.ShapeDtypeStruct((NW, NP), jnp.float32),
        mesh=_mesh(),
        compiler_params=pltpu.CompilerParams(needs_layout_passes=False),
        scratch_types=[
            pltpu.VMEM((CPT, CHUNK), jnp.int32),
            pltpu.VMEM((CPT, CHUNK), jnp.int32),
            pltpu.VMEM((NP,), jnp.float32),
        ],
    )
    def hist_kernel(rows_hbm, colp_hbm, h_hbm, ridx, cidx, hist):
        cid = lax.axis_index("c")
        sid = lax.axis_index("s")
        wid = cid * NS + sid
        pltpu.sync_copy(rows_hbm.at[pl.ds(wid * CPT, CPT)], ridx)
        pltpu.sync_copy(colp_hbm.at[pl.ds(wid * CPT, CPT)], cidx)
        zeros16 = jnp.zeros((L,), jnp.float32)

        def zbody(i, carry):
            hist[pl.ds(i * L, L)] = zeros16
            return carry

        lax.fori_loop(0, NP // L, zbody, 0)
        ones16 = jnp.ones((L,), jnp.float32)
        nvec = jnp.full((L,), N, jnp.int32)

        def cbody(c, carry):
            for i in range(CHUNK // L):
                rv = ridx[c, pl.ds(i * L, L)]
                cv = cidx[c, pl.ds(i * L, L)]
                rv = jnp.where(cv < nvec, rv, nvec)
                plsc.addupdate_scatter(hist, [rv], ones16)
            return carry

        lax.fori_loop(0, CPT, cbody, 0)
        pltpu.sync_copy(hist, h_hbm.at[wid])

    return hist_kernel(rows2d, colp2d)


def _sc_edge(z, rows2d, colp2d):
    @functools.partial(
        pl.kernel,
        out_type=jax.ShapeDtypeStruct((NC, NP, D), jnp.float32),
        mesh=_mesh(),
        compiler_params=pltpu.CompilerParams(needs_layout_passes=False),
        scratch_types=[
            pltpu.VMEM((CPT, CHUNK), jnp.int32),
            pltpu.VMEM((CPT, CHUNK), jnp.int32),
            pltpu.VMEM((CHUNK, D), jnp.float32),
            pltpu.VMEM_SHARED((NP, D), jnp.float32),
            pltpu.SemaphoreType.DMA,
        ],
    )
    def edge_kernel(z_hbm, rows_hbm, colp_hbm, p_hbm, ridx, cidx, buf, acc, sem):
        cid = lax.axis_index("c")
        sid = lax.axis_index("s")
        wid = cid * NS + sid
        pltpu.sync_copy(rows_hbm.at[pl.ds(wid * CPT, CPT)], ridx)
        pltpu.sync_copy(colp_hbm.at[pl.ds(wid * CPT, CPT)], cidx)
        zeros16 = jnp.zeros((L,), jnp.float32)

        def zbody(j, carry):
            for i in range(D // L):
                buf[j, pl.ds(i * L, L)] = zeros16
            return carry

        lax.fori_loop(0, CHUNK, zbody, 0)
        for j in range(RPS // CHUNK):
            pltpu.sync_copy(buf, acc.at[pl.ds(sid * RPS + j * CHUNK, CHUNK)])
        plsc.subcore_barrier()

        def ebody(c, carry):
            pltpu.async_copy(z_hbm.at[ridx.at[c]], buf, sem).wait()
            pltpu.sync_copy(buf, acc.at[cidx.at[c]], add=True)
            return carry

        lax.fori_loop(0, CPT, ebody, 0)
        plsc.subcore_barrier()
        pltpu.sync_copy(
            acc.at[pl.ds(sid * RPS, RPS)], p_hbm.at[cid, pl.ds(sid * RPS, RPS)]
        )

    return edge_kernel(z, rows2d, colp2d)


def _tc_init(h, x_pad):
    def body(h_ref, x_ref, z_ref, dis_ref):
        counts = jnp.sum(h_ref[...], axis=0)
        dis = lax.rsqrt(counts + 1.0)[:, None]
        z_ref[...] = dis * x_ref[...]
        dis_ref[...] = dis

    return pl.pallas_call(
        body,
        grid=(NP // BLK,),
        in_specs=[
            pl.BlockSpec((NW, BLK), lambda i: (0, i)),
            pl.BlockSpec((BLK, D), lambda i: (i, 0)),
        ],
        out_specs=[
            pl.BlockSpec((BLK, D), lambda i: (i, 0)),
            pl.BlockSpec((BLK, 1), lambda i: (i, 0)),
        ],
        out_shape=[
            jax.ShapeDtypeStruct((NP, D), jnp.float32),
            jax.ShapeDtypeStruct((NP, 1), jnp.float32),
        ],
    )(h, x_pad)


def _tc_combine(p, z, dis):
    def body(p_ref, z_ref, dis_ref, out_ref, z2_ref):
        s = p_ref[0] + p_ref[1] + z_ref[...]
        dd = dis_ref[...]
        o = dd * s
        out_ref[...] = o
        z2_ref[...] = dd * o

    return pl.pallas_call(
        body,
        grid=(NP // BLK,),
        in_specs=[
            pl.BlockSpec((NC, BLK, D), lambda i: (0, i, 0)),
            pl.BlockSpec((BLK, D), lambda i: (i, 0)),
            pl.BlockSpec((BLK, 1), lambda i: (i, 0)),
        ],
        out_specs=[
            pl.BlockSpec((BLK, D), lambda i: (i, 0)),
            pl.BlockSpec((BLK, D), lambda i: (i, 0)),
        ],
        out_shape=[
            jax.ShapeDtypeStruct((NP, D), jnp.float32),
            jax.ShapeDtypeStruct((NP, D), jnp.float32),
        ],
    )(p, z, dis)


def kernel(feature, edge_index):
    row = edge_index[0]
    col = edge_index[1]
    colp = jnp.where(row == col, N, col).astype(jnp.int32)
    pad = EP - row.shape[0]
    rows_p = jnp.concatenate([row.astype(jnp.int32), jnp.full((pad,), N, jnp.int32)])
    colp_p = jnp.concatenate([colp, jnp.full((pad,), N, jnp.int32)])
    rows2d = rows_p.reshape(EP // CHUNK, CHUNK)
    colp2d = colp_p.reshape(EP // CHUNK, CHUNK)
    x_pad = jnp.pad(feature, ((0, NP - N), (0, 0)))

    h = _sc_hist(rows2d, colp2d)
    z, dis = _tc_init(h, x_pad)
    outs = [feature]
    for _ in range(K):
        p = _sc_edge(z, rows2d, colp2d)
        o, z = _tc_combine(p, z, dis)
        outs.append(o[:N])
    return jnp.concatenate(outs, axis=1)


# speedup vs baseline: 5.3161x; 1.1197x over previous
import functools

import jax
import jax.numpy as jnp
from jax import lax
from jax.experimental import pallas as pl
from jax.experimental.pallas import tpu as pltpu
from jax.experimental.pallas import tpu_sc as plsc

N = 10000
D = 128
K = 8
NP = 10240
NC, NS, L = 2, 16, 16
NW = NC * NS
CHUNK = 128
CPT = 80
EP = NW * CPT * CHUNK
RPS = NP // NS
SLOT = 8
BLK = 256


def _mesh():
    return plsc.VectorSubcoreMesh(core_axis_name="c", subcore_axis_name="s")


def _sc_hist(rows2d, colp2d):
    @functools.partial(
        pl.kernel,
        out_type=jax.ShapeDtypeStruct((NW, NP), jnp.float32),
        mesh=_mesh(),
        compiler_params=pltpu.CompilerParams(needs_layout_passes=False),
        scratch_types=[
            pltpu.VMEM((CPT, CHUNK), jnp.int32),
            pltpu.VMEM((CPT, CHUNK), jnp.int32),
            pltpu.VMEM((NP,), jnp.float32),
        ],
    )
    def hist_kernel(rows_hbm, colp_hbm, h_hbm, ridx, cidx, hist):
        cid = lax.axis_index("c")
        sid = lax.axis_index("s")
        wid = cid * NS + sid
        pltpu.sync_copy(rows_hbm.at[pl.ds(wid * CPT, CPT)], ridx)
        pltpu.sync_copy(colp_hbm.at[pl.ds(wid * CPT, CPT)], cidx)
        zeros16 = jnp.zeros((L,), jnp.float32)

        def zbody(i, carry):
            hist[pl.ds(i * L, L)] = zeros16
            return carry

        lax.fori_loop(0, NP // L, zbody, 0)
        ones16 = jnp.ones((L,), jnp.float32)
        nvec = jnp.full((L,), N, jnp.int32)

        def cbody(c, carry):
            for i in range(CHUNK // L):
                rv = ridx[c, pl.ds(i * L, L)]
                cv = cidx[c, pl.ds(i * L, L)]
                rv = jnp.where(cv < nvec, rv, nvec)
                plsc.addupdate_scatter(hist, [rv], ones16)
            return carry

        lax.fori_loop(0, CPT, cbody, 0)
        pltpu.sync_copy(hist, h_hbm.at[wid])

    return hist_kernel(rows2d, colp2d)


def _sc_edge(z, rows2d, colp2d):
    @functools.partial(
        pl.kernel,
        out_type=jax.ShapeDtypeStruct((NC, NP, D), jnp.float32),
        mesh=_mesh(),
        compiler_params=pltpu.CompilerParams(needs_layout_passes=False),
        scratch_types=[
            pltpu.VMEM((SLOT, CHUNK), jnp.int32),
            pltpu.VMEM((SLOT, CHUNK), jnp.int32),
            pltpu.VMEM((SLOT, CHUNK), jnp.int32),
            pltpu.VMEM((SLOT, CHUNK), jnp.int32),
            pltpu.VMEM((CHUNK, D), jnp.float32),
            pltpu.VMEM((CHUNK, D), jnp.float32),
            pltpu.VMEM_SHARED((NP, D), jnp.float32),
            pltpu.SemaphoreType.DMA,
            pltpu.SemaphoreType.DMA,
            pltpu.SemaphoreType.DMA,
            pltpu.SemaphoreType.DMA,
        ],
    )
    def edge_kernel(
        z_hbm, rows_hbm, colp_hbm, p_hbm,
        rA, rB, cA, cB, buf0, buf1, acc, sA, sB, g0, g1,
    ):
        cid = lax.axis_index("c")
        sid = lax.axis_index("s")
        wid = cid * NS + sid
        base = wid * CPT
        bufs = (buf0, buf1)
        gsems = (g0, g1)
        rslots = (rA, rB)
        cslots = (cA, cB)

        def load_slot(rs, cs, sem, off):
            pltpu.async_copy(rows_hbm.at[pl.ds(off, SLOT)], rs, sem)
            pltpu.async_copy(colp_hbm.at[pl.ds(off, SLOT)], cs, sem)

        def wait_slot(rs, cs, sem):
            pltpu.make_async_copy(rows_hbm.at[pl.ds(base, SLOT)], rs, sem).wait()
            pltpu.make_async_copy(colp_hbm.at[pl.ds(base, SLOT)], cs, sem).wait()

        def wait_gather(b):
            pltpu.make_async_copy(z_hbm.at[rA.at[0]], bufs[b], gsems[b]).wait()

        zeros16 = jnp.zeros((L,), jnp.float32)

        def zbody(j, carry):
            for i in range(D // L):
                buf0[j, pl.ds(i * L, L)] = zeros16
            return carry

        lax.fori_loop(0, CHUNK, zbody, 0)
        for j in range(RPS // CHUNK):
            pltpu.sync_copy(buf0, acc.at[pl.ds(sid * RPS + j * CHUNK, CHUNK)])
        load_slot(rA, cA, sA, base)
        wait_slot(rA, cA, sA)
        pltpu.async_copy(z_hbm.at[rA.at[0]], buf0, g0)
        pltpu.async_copy(z_hbm.at[rA.at[1]], buf1, g1)
        plsc.subcore_barrier()

        def phase(i, last):
            c0 = i * 2 * SLOT
            for b in range(2 * SLOT):
                gb = b % 2
                if not last:
                    if b == 0:
                        load_slot(rB, cB, sB, base + c0 + SLOT)
                    if b == 8:
                        load_slot(rA, cA, sA, base + c0 + 2 * SLOT)
                elif b == 0:
                    load_slot(rB, cB, sB, base + c0 + SLOT)
                if b == 6:
                    wait_slot(rB, cB, sB)
                if b == 14 and not last:
                    wait_slot(rA, cA, sA)
                wait_gather(gb)
                cs = cslots[b // SLOT]
                pltpu.sync_copy(bufs[gb], acc.at[cs.at[b % SLOT]], add=True)
                b2 = b + 2
                if last and b2 >= 2 * SLOT:
                    continue
                rs2 = rslots[(b2 // SLOT) % 2]
                pltpu.async_copy(z_hbm.at[rs2.at[b2 % SLOT]], bufs[gb], gsems[gb])

        def pbody(i, carry):
            phase(i, last=False)
            return carry

        lax.fori_loop(0, CPT // (2 * SLOT) - 1, pbody, 0)
        phase(CPT // (2 * SLOT) - 1, last=True)
        plsc.subcore_barrier()
        pltpu.sync_copy(
            acc.at[pl.ds(sid * RPS, RPS)], p_hbm.at[cid, pl.ds(sid * RPS, RPS)]
        )

    return edge_kernel(z, rows2d, colp2d)


def _tc_init(h, x_pad):
    def body(h_ref, x_ref, z_ref, dis_ref):
        counts = jnp.sum(h_ref[...], axis=0)
        dis = lax.rsqrt(counts + 1.0)[:, None]
        z_ref[...] = dis * x_ref[...]
        dis_ref[...] = dis

    return pl.pallas_call(
        body,
        grid=(NP // BLK,),
        in_specs=[
            pl.BlockSpec((NW, BLK), lambda i: (0, i)),
            pl.BlockSpec((BLK, D), lambda i: (i, 0)),
        ],
        out_specs=[
            pl.BlockSpec((BLK, D), lambda i: (i, 0)),
            pl.BlockSpec((BLK, 1), lambda i: (i, 0)),
        ],
        out_shape=[
            jax.ShapeDtypeStruct((NP, D), jnp.float32),
            jax.ShapeDtypeStruct((NP, 1), jnp.float32),
        ],
    )(h, x_pad)


def _tc_combine(p, z, dis):
    def body(p_ref, z_ref, dis_ref, out_ref, z2_ref):
        s = p_ref[0] + p_ref[1] + z_ref[...]
        dd = dis_ref[...]
        o = dd * s
        out_ref[...] = o
        z2_ref[...] = dd * o

    return pl.pallas_call(
        body,
        grid=(NP // BLK,),
        in_specs=[
            pl.BlockSpec((NC, BLK, D), lambda i: (0, i, 0)),
            pl.BlockSpec((BLK, D), lambda i: (i, 0)),
            pl.BlockSpec((BLK, 1), lambda i: (i, 0)),
        ],
        out_specs=[
            pl.BlockSpec((BLK, D), lambda i: (i, 0)),
            pl.BlockSpec((BLK, D), lambda i: (i, 0)),
        ],
        out_shape=[
            jax.ShapeDtypeStruct((NP, D), jnp.float32),
            jax.ShapeDtypeStruct((NP, D), jnp.float32),
        ],
    )(p, z, dis)


def kernel(feature, edge_index):
    row = edge_index[0]
    col = edge_index[1]
    colp = jnp.where(row == col, N, col).astype(jnp.int32)
    pad = EP - row.shape[0]
    rows_p = jnp.concatenate([row.astype(jnp.int32), jnp.full((pad,), N, jnp.int32)])
    colp_p = jnp.concatenate([colp, jnp.full((pad,), N, jnp.int32)])
    rows2d = rows_p.reshape(EP // CHUNK, CHUNK)
    colp2d = colp_p.reshape(EP // CHUNK, CHUNK)
    x_pad = jnp.pad(feature, ((0, NP - N), (0, 0)))

    h = _sc_hist(rows2d, colp2d)
    z, dis = _tc_init(h, x_pad)
    outs = [feature]
    for _ in range(K):
        p = _sc_edge(z, rows2d, colp2d)
        o, z = _tc_combine(p, z, dis)
        outs.append(o[:N])
    return jnp.concatenate(outs, axis=1)
